# baseline (speedup 1.0000x reference)
import functools

import jax
import jax.numpy as jnp
from jax import lax
from jax.experimental import pallas as pl
from jax.experimental.pallas import tpu as pltpu
from jax.experimental.pallas import tpu_sc as plsc

_LANES = 16
_SUB = 8
_NSET = 4
_PD = 2


def _emb_lookup(idx, word_emb, pos_emb, num_cores, num_subcores):
    B, T = idx.shape
    V, D = word_emb.shape
    NW = num_cores * num_subcores
    CH = T // NW
    NSU = CH // _SUB

    mesh = plsc.VectorSubcoreMesh(core_axis_name="c", subcore_axis_name="s")

    @functools.partial(
        pl.kernel,
        mesh=mesh,
        out_type=jax.ShapeDtypeStruct((B, T, D), jnp.float32),
        scratch_types=[
            pltpu.VMEM((B, CH), jnp.int32),
        ] + [pltpu.VMEM((_SUB, D), jnp.float32)] * ((B + 1) * _NSET)
          + [pltpu.SemaphoreType.DMA] * (3 * _NSET + 1),
    )
    def emb_kernel(idx_hbm, word_hbm, pos_hbm, out_hbm, idx_v, *bufs_and_sems):
        nb = B + 1
        sets = [list(bufs_and_sems[m * nb:(m + 1) * nb]) for m in range(_NSET)]
        rest = bufs_and_sems[nb * _NSET:]
        gsem = list(rest[:_NSET])
        osem = list(rest[_NSET:2 * _NSET])
        psem = list(rest[2 * _NSET:3 * _NSET])
        isem = rest[3 * _NSET]

        wid = lax.axis_index("s") * num_cores + lax.axis_index("c")
        t0 = wid * CH
        icopies = [
            pltpu.async_copy(idx_hbm.at[b, pl.ds(t0, CH)], idx_v.at[b], isem)
            for b in range(B)
        ]

        fills = [None] * NSU
        outs = [None] * NSU

        def fill(su):
            m = su % _NSET
            pcopy = pltpu.async_copy(
                pos_hbm.at[pl.ds(t0 + su * _SUB, _SUB)], sets[m][B], psem[m])
            gcopies = [
                pltpu.async_copy(
                    word_hbm.at[idx_v.at[b, pl.ds(su * _SUB, _SUB)]],
                    sets[m][b], gsem[m])
                for b in range(B)
            ]
            fills[su] = gcopies + [pcopy]

        def outw(su):
            m = su % _NSET
            outs[su] = [
                pltpu.async_copy(
                    sets[m][b], out_hbm.at[b, pl.ds(t0 + su * _SUB, _SUB)],
                    osem[m])
                for b in range(B)
            ]

        for c in icopies:
            c.wait()
        for su in range(_PD):
            fill(su)
        for su in range(NSU):
            for c in fills[su]:
                c.wait()
            m = su % _NSET
            bufs = sets[m]

            @plsc.parallel_loop(0, _SUB, unroll=1)
            def add_row(i, _bufs=bufs):
                for k in range(D // _LANES):
                    sl = pl.ds(k * _LANES, _LANES)
                    p = _bufs[B][i, sl]
                    for b in range(B):
                        plsc.addupdate(_bufs[b].at[i, sl], p)

            outw(su)
            if su + _PD < NSU:
                back = su - (_NSET - _PD)
                if back >= 0:
                    for c in outs[back]:
                        c.wait()
                fill(su + _PD)
        for su in range(max(0, NSU - _NSET), NSU):
            for c in outs[su]:
                c.wait()

    return emb_kernel(idx, word_emb, pos_emb)


def kernel(idx, word_emb, pos_emb):
    idx = jnp.asarray(idx, jnp.int32)
    return _emb_lookup(idx, word_emb, pos_emb, num_cores=2, num_subcores=16)


# speedup vs baseline: 1.0118x; 1.0118x over previous
import functools

import jax
import jax.numpy as jnp
from jax import lax
from jax.experimental import pallas as pl
from jax.experimental.pallas import tpu as pltpu
from jax.experimental.pallas import tpu_sc as plsc

_LANES = 16
_SUB = 8
_NSET = 4
_PD = 3


def _emb_lookup(idx, word_emb, pos_emb, num_cores, num_subcores):
    B, T = idx.shape
    V, D = word_emb.shape
    NW = num_cores * num_subcores
    CH = T // NW
    NSU = CH // _SUB

    mesh = plsc.VectorSubcoreMesh(core_axis_name="c", subcore_axis_name="s")

    @functools.partial(
        pl.kernel,
        mesh=mesh,
        out_type=jax.ShapeDtypeStruct((B, T, D), jnp.float32),
        scratch_types=[
            pltpu.VMEM((B, CH), jnp.int32),
        ] + [pltpu.VMEM((_SUB, D), jnp.float32)] * ((B + 1) * _NSET)
          + [pltpu.SemaphoreType.DMA] * (3 * _NSET + 1),
    )
    def emb_kernel(idx_hbm, word_hbm, pos_hbm, out_hbm, idx_v, *bufs_and_sems):
        nb = B + 1
        sets = [list(bufs_and_sems[m * nb:(m + 1) * nb]) for m in range(_NSET)]
        rest = bufs_and_sems[nb * _NSET:]
        gsem = list(rest[:_NSET])
        osem = list(rest[_NSET:2 * _NSET])
        psem = list(rest[2 * _NSET:3 * _NSET])
        isem = rest[3 * _NSET]

        wid = lax.axis_index("s") * num_cores + lax.axis_index("c")
        t0 = wid * CH
        icopies = [
            pltpu.async_copy(idx_hbm.at[b, pl.ds(t0, CH)], idx_v.at[b], isem)
            for b in range(B)
        ]

        fills = [None] * NSU
        outs = [None] * NSU

        def fill(su):
            m = su % _NSET
            pcopy = pltpu.async_copy(
                pos_hbm.at[pl.ds(t0 + su * _SUB, _SUB)], sets[m][B], psem[m])
            gcopies = [
                pltpu.async_copy(
                    word_hbm.at[idx_v.at[b, pl.ds(su * _SUB, _SUB)]],
                    sets[m][b], gsem[m])
                for b in range(B)
            ]
            fills[su] = gcopies + [pcopy]

        def outw(su):
            m = su % _NSET
            outs[su] = [
                pltpu.async_copy(
                    sets[m][b], out_hbm.at[b, pl.ds(t0 + su * _SUB, _SUB)],
                    osem[m])
                for b in range(B)
            ]

        for c in icopies:
            c.wait()
        for su in range(_PD):
            fill(su)
        for su in range(NSU):
            for c in fills[su]:
                c.wait()
            m = su % _NSET
            bufs = sets[m]

            @plsc.parallel_loop(0, _SUB, unroll=1)
            def add_row(i, _bufs=bufs):
                for k in range(D // _LANES):
                    sl = pl.ds(k * _LANES, _LANES)
                    p = _bufs[B][i, sl]
                    for b in range(B):
                        plsc.addupdate(_bufs[b].at[i, sl], p)

            outw(su)
            if su + _PD < NSU:
                back = su - (_NSET - _PD)
                if back >= 0:
                    for c in outs[back]:
                        c.wait()
                fill(su + _PD)
        for su in range(max(0, NSU - _NSET), NSU):
            for c in outs[su]:
                c.wait()

    return emb_kernel(idx, word_emb, pos_emb)


def kernel(idx, word_emb, pos_emb):
    idx = jnp.asarray(idx, jnp.int32)
    return _emb_lookup(idx, word_emb, pos_emb, num_cores=2, num_subcores=16)
